# Initial kernel scaffold; baseline (speedup 1.0000x reference)
import jax
import jax.numpy as jnp
from jax.experimental import pallas as pl


def kernel(X, edge_index, cluster_assignment, full_edge_attr, W, b):
    raise NotImplementedError("write your pallas kernel here")


# speedup vs baseline: 54.8478x; 54.8478x over previous
import functools

import jax
import jax.numpy as jnp
from jax import lax
from jax.experimental import pallas as pl
from jax.experimental.pallas import tpu as pltpu
from jax.experimental.pallas import tpu_sc as plsc

N = 10000
E = 320000
D = 128
NC = 2
NS = 16
NW = NC * NS
EPT = E // NW
NP = 10240
SLC = NP // NS
CAP = 10240
K = 256
RB = 400
GR = N // RB


def _edge_kernel(src_hbm, dst_hbm, clu_hbm,
                 deg_hbm, cnt_hbm, acts_hbm, actd_hbm, counts_hbm,
                 clu_v, src_v, dst_v, wgt_v, cs_v, act_s, act_d, cbuf,
                 idb, cnt_loc, deg_sp, cnt_sp):
    cid = lax.axis_index("c")
    sid = lax.axis_index("s")
    w = sid * NC + cid
    base = w * EPT
    lanes = lax.iota(jnp.int32, 16)
    z16f = jnp.zeros((16,), jnp.float32)

    pltpu.sync_copy(clu_hbm, clu_v)
    pltpu.sync_copy(src_hbm.at[pl.ds(base, EPT)], src_v)
    pltpu.sync_copy(dst_hbm.at[pl.ds(base, EPT)], dst_v)

    def zero_w(i, c):
        wgt_v[pl.ds(i * 16, 16)] = z16f
        return c
    lax.fori_loop(0, SLC // 16, zero_w, 0)
    for j in range(8):
        cnt_loc[pl.ds(j * 16, 16)] = z16f
        idb[pl.ds(j * 16, 16)] = j * 16 + lanes

    pltpu.sync_copy(wgt_v.at[pl.ds(0, SLC)], deg_sp.at[pl.ds(sid * SLC, SLC)])

    @pl.when(sid == 0)
    def _():
        pltpu.sync_copy(cnt_loc, cnt_sp)

    plsc.subcore_barrier()

    ones16 = jnp.ones((16,), jnp.float32)

    def body(i, cnt):
        off = i * 16
        s16 = src_v[pl.ds(off, 16)]
        d16 = dst_v[pl.ds(off, 16)]
        cs = plsc.load_gather(clu_v, [s16])
        cd = plsc.load_gather(clu_v, [d16])
        m = cs == cd
        mi = jnp.where(m, 1, 0)
        wgt_v[pl.ds(off, 16)] = jnp.where(m, 1.0, 0.0)
        cs_v[pl.ds(off, 16)] = cs
        plsc.store_scatter(cnt_loc, [cs], ones16, mask=m)
        pos = cnt + jnp.cumsum(mi) - 1
        plsc.store_scatter(act_s, [pos], s16, mask=m)
        plsc.store_scatter(act_d, [pos], d16, mask=m)
        return cnt + jnp.sum(mi)

    cnt = lax.fori_loop(0, EPT // 16, body, jnp.int32(0))

    cbuf[pl.ds(0, 16)] = jnp.where(lanes == 0, cnt, 0)
    pltpu.sync_copy(cbuf, counts_hbm.at[w])
    pltpu.sync_copy(act_s, acts_hbm.at[pl.ds(w * CAP, CAP)])
    pltpu.sync_copy(act_d, actd_hbm.at[pl.ds(w * CAP, CAP)])

    pltpu.sync_copy(wgt_v, deg_sp.at[dst_v], add=True)
    pltpu.sync_copy(cnt_loc, cnt_sp.at[idb], add=True)

    plsc.subcore_barrier()

    pltpu.sync_copy(deg_sp.at[pl.ds(sid * SLC, SLC)],
                    deg_hbm.at[pl.ds(cid * NP + sid * SLC, SLC)])

    @pl.when(sid == 0)
    def _():
        pltpu.sync_copy(cnt_sp, cnt_hbm.at[pl.ds(cid * 128, 128)])


_edge_call = functools.partial(
    pl.kernel,
    mesh=plsc.VectorSubcoreMesh(core_axis_name="c", subcore_axis_name="s"),
    compiler_params=pltpu.CompilerParams(needs_layout_passes=False),
    out_type=[
        jax.ShapeDtypeStruct((NC * NP,), jnp.float32),
        jax.ShapeDtypeStruct((NC * 128,), jnp.float32),
        jax.ShapeDtypeStruct((NW * CAP,), jnp.int32),
        jax.ShapeDtypeStruct((NW * CAP,), jnp.int32),
        jax.ShapeDtypeStruct((NW, 16), jnp.int32),
    ],
    scratch_types=[
        pltpu.VMEM((N,), jnp.int32),
        pltpu.VMEM((EPT,), jnp.int32),
        pltpu.VMEM((EPT,), jnp.int32),
        pltpu.VMEM((EPT,), jnp.float32),
        pltpu.VMEM((EPT,), jnp.int32),
        pltpu.VMEM((CAP,), jnp.int32),
        pltpu.VMEM((CAP,), jnp.int32),
        pltpu.VMEM((16,), jnp.int32),
        pltpu.VMEM((128,), jnp.int32),
        pltpu.VMEM((128,), jnp.float32),
        pltpu.VMEM_SHARED((NP,), jnp.float32),
        pltpu.VMEM_SHARED((128,), jnp.float32),
    ],
)(_edge_kernel)


def _gather_scatter_kernel(y_hbm, acts_hbm, actd_hbm, counts_hbm,
                           acc_hbm, sidx, didx, rows, cbuf, acc_sp, sem):
    cid = lax.axis_index("c")
    sid = lax.axis_index("s")
    w = sid * NC + cid
    lanes = lax.iota(jnp.int32, 16)
    z16f = jnp.zeros((16,), jnp.float32)

    def zrow(r, c):
        for j in range(D // 16):
            rows[r, pl.ds(j * 16, 16)] = z16f
        return c
    lax.fori_loop(0, K, zrow, 0)

    base_r = sid * SLC
    pltpu.sync_copy(rows, acc_sp.at[pl.ds(base_r, K)])
    pltpu.sync_copy(rows, acc_sp.at[pl.ds(base_r + K, K)])
    pltpu.sync_copy(rows.at[pl.ds(0, SLC - 2 * K)],
                    acc_sp.at[pl.ds(base_r + 2 * K, SLC - 2 * K)])
    plsc.subcore_barrier()

    pltpu.sync_copy(counts_hbm.at[w], cbuf)
    n = jnp.sum(cbuf[pl.ds(0, 16)])
    nch = (n + K - 1) // K

    def chunk(ch, c):
        off = ch * K
        pltpu.sync_copy(acts_hbm.at[pl.ds(w * CAP + off, K)], sidx)
        pltpu.sync_copy(actd_hbm.at[pl.ds(w * CAP + off, K)], didx)
        for j in range(K // 16):
            gp = off + j * 16 + lanes
            valid = gp < n
            s16 = sidx[pl.ds(j * 16, 16)]
            d16 = didx[pl.ds(j * 16, 16)]
            sidx[pl.ds(j * 16, 16)] = jnp.where(valid, s16, 0)
            didx[pl.ds(j * 16, 16)] = jnp.where(valid, d16, N + lanes)
        pltpu.async_copy(y_hbm.at[sidx], rows, sem).wait()
        pltpu.sync_copy(rows, acc_sp.at[didx], add=True)
        return c

    lax.fori_loop(0, nch, chunk, 0)
    plsc.subcore_barrier()

    pltpu.sync_copy(acc_sp.at[pl.ds(base_r, SLC)],
                    acc_hbm.at[pl.ds(cid * NP + base_r, SLC)])


_gather_scatter_call = functools.partial(
    pl.kernel,
    mesh=plsc.VectorSubcoreMesh(core_axis_name="c", subcore_axis_name="s"),
    compiler_params=pltpu.CompilerParams(needs_layout_passes=False),
    out_type=[jax.ShapeDtypeStruct((NC * NP, D), jnp.float32)],
    scratch_types=[
        pltpu.VMEM((K,), jnp.int32),
        pltpu.VMEM((K,), jnp.int32),
        pltpu.VMEM((K, D), jnp.float32),
        pltpu.VMEM((16,), jnp.int32),
        pltpu.VMEM_SHARED((NP, D), jnp.float32),
        pltpu.SemaphoreType.DMA,
    ],
)(_gather_scatter_kernel)


def _mm_kernel(x_ref, w_ref, degp_ref, cntp_ref, y_ref, dinv_ref, cnt_ref):
    deg = degp_ref[0] + degp_ref[1] + 1.0
    dinv = lax.rsqrt(deg)
    xw = jnp.dot(x_ref[...], w_ref[...], preferred_element_type=jnp.float32)
    y_ref[...] = dinv * xw
    dinv_ref[...] = dinv
    cnt_ref[...] = cntp_ref[0:1] + cntp_ref[1:2]


def _mm_call(x, w, degp, cntp):
    return pl.pallas_call(
        _mm_kernel,
        grid=(GR,),
        in_specs=[
            pl.BlockSpec((RB, D), lambda i: (i, 0)),
            pl.BlockSpec((D, D), lambda i: (0, 0)),
            pl.BlockSpec((NC, RB, 1), lambda i: (0, i, 0)),
            pl.BlockSpec((NC, 128), lambda i: (0, 0)),
        ],
        out_specs=[
            pl.BlockSpec((RB, D), lambda i: (i, 0)),
            pl.BlockSpec((RB, 1), lambda i: (i, 0)),
            pl.BlockSpec((1, 128), lambda i: (0, 0)),
        ],
        out_shape=[
            jax.ShapeDtypeStruct((N, D), jnp.float32),
            jax.ShapeDtypeStruct((N, 1), jnp.float32),
            jax.ShapeDtypeStruct((1, 128), jnp.float32),
        ],
    )(x, w, degp, cntp)


def _comb_kernel(a0_ref, a1_ref, y_ref, dinv_ref, clu_ref, cnt_ref,
                 x_ref, b_ref, out_ref):
    acc = a0_ref[...] + a1_ref[...] + y_ref[...]
    o = dinv_ref[...] * acc + b_ref[...]
    colid = lax.broadcasted_iota(jnp.int32, (RB, 128), 1)
    hit = jnp.logical_and(clu_ref[...] == colid, cnt_ref[...] > 0.0)
    upd = jnp.any(hit, axis=1, keepdims=True)
    out_ref[...] = jnp.where(upd, o, x_ref[...])


def _comb_call(a0, a1, y, dinv, clu, cnt, x, b):
    rb_spec = pl.BlockSpec((RB, D), lambda i: (i, 0))
    return pl.pallas_call(
        _comb_kernel,
        grid=(GR,),
        in_specs=[
            rb_spec, rb_spec, rb_spec,
            pl.BlockSpec((RB, 1), lambda i: (i, 0)),
            pl.BlockSpec((RB, 1), lambda i: (i, 0)),
            pl.BlockSpec((1, 128), lambda i: (0, 0)),
            rb_spec,
            pl.BlockSpec((1, 128), lambda i: (0, 0)),
        ],
        out_specs=rb_spec,
        out_shape=jax.ShapeDtypeStruct((N, D), jnp.float32),
    )(a0, a1, y, dinv, clu, cnt, x, b)


def kernel(X, edge_index, cluster_assignment, full_edge_attr, W, b):
    del full_edge_attr
    src = edge_index[0]
    dst = edge_index[1]
    deg_p, cnt_p, acts, actd, counts = _edge_call(src, dst,
                                                  cluster_assignment)
    y, dinv, cnt_tot = _mm_call(X, W,
                                deg_p.reshape(NC, NP, 1),
                                cnt_p.reshape(NC, 128))
    (acc,) = _gather_scatter_call(y, acts, actd, counts)
    a0 = acc[:N]
    a1 = acc[NP:NP + N]
    return _comb_call(a0, a1, y, dinv,
                      cluster_assignment.reshape(N, 1).astype(jnp.int32),
                      cnt_tot, X, b.reshape(1, 128))
